# baseline (speedup 1.0000x reference)
import functools

import jax
import jax.numpy as jnp
from jax import lax
from jax.experimental import pallas as pl
from jax.experimental.pallas import tpu as pltpu
from jax.experimental.pallas import tpu_sc as plsc

_IOU_THRESH = 0.1
_K = 10.0
_XY = 0.2
_WH = 0.1
_CLS = 0.35
_CONF = 0.35
_NG = (13, 26, 52)
_B = 16
_NCLS = 80
_ANCH = (
    ((3.625, 2.8125), (4.875, 6.1875), (11.65625, 10.1875)),
    ((1.875, 2.40625), (2.9375, 6.09375), (5.59375, 5.03125)),
    ((1.25, 1.625), (2.0, 3.75), (4.125, 2.875)),
)
_NT = 128
_CH = 85
_P1_CB = 4
_P2_CB = 1
_P2_NC = _B // _P2_CB

_sc_mesh = plsc.VectorSubcoreMesh(core_axis_name="c", subcore_axis_name="s")


@functools.partial(
    pl.kernel,
    mesh=_sc_mesh,
    out_type=[
        jax.ShapeDtypeStruct((3, _NT), jnp.int32),
        jax.ShapeDtypeStruct((3, _NT), jnp.int32),
        jax.ShapeDtypeStruct((3, _NT), jnp.int32),
        jax.ShapeDtypeStruct((3, _NT), jnp.int32),
        jax.ShapeDtypeStruct((3, _NT), jnp.int32),
        jax.ShapeDtypeStruct((3, _NT), jnp.float32),
    ],
    scratch_types=[
        pltpu.VMEM((6, _NT), jnp.float32),
        pltpu.VMEM((32,), jnp.int32),
        pltpu.VMEM((32,), jnp.int32),
        pltpu.VMEM((32,), jnp.int32),
        pltpu.VMEM((32,), jnp.int32),
        pltpu.VMEM((32,), jnp.int32),
        pltpu.VMEM((32,), jnp.float32),
    ],
)
def _sc_route(tT_hbm, b_out, a_out, gj_out, gi_out, lin_out, iou_out,
              tT_v, b_v, a_v, gj_v, gi_v, lin_v, iou_v):
    wid = lax.axis_index("s") * 2 + lax.axis_index("c")
    j0 = (wid % 4) * 32
    for sc in range(3):
        @pl.when(wid // 4 == sc)
        def _(sc=sc):
            G = float(_NG[sc])
            Gi = _NG[sc]
            anch = _ANCH[sc]
            pltpu.sync_copy(tT_hbm, tT_v)
            for chunk in range(2):
                base = j0 + chunk * 16
                bf = tT_v[0, pl.ds(base, 16)]
                xf = tT_v[2, pl.ds(base, 16)]
                yf = tT_v[3, pl.ds(base, 16)]
                wf = tT_v[4, pl.ds(base, 16)]
                hf = tT_v[5, pl.ds(base, 16)]
                gw = wf * G
                gh = hf * G
                ious = []
                for k in range(3):
                    aw, ah = anch[k]
                    inter = jnp.minimum(aw, gw) * jnp.minimum(ah, gh)
                    union = aw * ah + gw * gh - inter + 1e-16
                    ious.append(inter / union)
                a16 = jnp.where(ious[1] > ious[0], 1, 0).astype(jnp.int32)
                best = jnp.maximum(ious[0], ious[1])
                a16 = jnp.where(ious[2] > best, 2, a16)
                best = jnp.maximum(best, ious[2])
                b16 = bf.astype(jnp.int32)
                gi16 = (xf * G).astype(jnp.int32)
                gj16 = (yf * G).astype(jnp.int32)
                dst = pl.ds(chunk * 16, 16)
                b_v[dst] = b16
                a_v[dst] = a16
                gj_v[dst] = gj16
                gi_v[dst] = gi16
                lin_v[dst] = (((b16 * 3 + a16) * Gi + gj16) * Gi) + gi16
                iou_v[dst] = best
            pltpu.sync_copy(b_v, b_out.at[sc, pl.ds(j0, 32)])
            pltpu.sync_copy(a_v, a_out.at[sc, pl.ds(j0, 32)])
            pltpu.sync_copy(gj_v, gj_out.at[sc, pl.ds(j0, 32)])
            pltpu.sync_copy(gi_v, gi_out.at[sc, pl.ds(j0, 32)])
            pltpu.sync_copy(lin_v, lin_out.at[sc, pl.ds(j0, 32)])
            pltpu.sync_copy(iou_v, iou_out.at[sc, pl.ds(j0, 32)])


def _softplus_sum(v):
    sp = jnp.maximum(v, 0.0) + jnp.log1p(jnp.exp(-jnp.abs(v)))
    lane = lax.broadcasted_iota(jnp.int32, v.shape, v.ndim - 1)
    return jnp.sum(jnp.where(lane == 4, sp, 0.0))


def _main_body(bS_ref, aS_ref, gjS_ref, giS_ref,
               t_ref, a_ref, lin_ref, iou_ref,
               p0h, p1h, p2h, out_ref,
               v0, v1a, v1b, v1c, v1d, v2a, v2b, v2c, v2d,
               gath, sem0, sem1, sem2, semr):
    ph = (p0h, p1h, p2h)
    bufs1 = (v1a, v1b, v1c, v1d)
    bufs2 = (v2a, v2b, v2c, v2d)

    def p1_copy(k, slot):
        return pltpu.make_async_copy(
            p1h.at[pl.ds(k * _P1_CB, _P1_CB)], bufs1[slot], sem1.at[slot])

    def p2_copy(k, slot):
        return pltpu.make_async_copy(
            p2h.at[pl.ds(k * _P2_CB, _P2_CB)], bufs2[slot], sem2.at[slot])

    def row_copy(s, t):
        return pltpu.make_async_copy(
            ph[s].at[bS_ref[s, t], aS_ref[s, t], gjS_ref[s, t],
                     pl.ds(giS_ref[s, t], 1), :],
            gath.at[s, pl.ds(t, 1), :],
            semr)

    for j in range(4):
        p2_copy(j, j).start()
    for j in range(4):
        p1_copy(j, j).start()
    c0 = pltpu.make_async_copy(p0h, v0, sem0)
    c0.start()

    for s in range(3):
        def rbody(t, _, s=s):
            row_copy(s, t).start()
            return 0
        lax.fori_loop(0, _NT, rbody, 0)

    acc2 = jnp.float32(0.0)
    for k in range(_P2_NC):
        p2_copy(k, k % 4).wait()
        acc2 = acc2 + _softplus_sum(bufs2[k % 4][...])
        if k + 4 < _P2_NC:
            p2_copy(k + 4, k % 4).start()

    acc1 = jnp.float32(0.0)
    for k in range(4):
        p1_copy(k, k).wait()
        acc1 = acc1 + _softplus_sum(bufs1[k][...])

    c0.wait()
    acc0 = _softplus_sum(v0[...])
    accs = (acc0, acc1, acc2)

    for s in range(3):
        def wbody(t, _, s=s):
            row_copy(s, t).wait()
            return 0
        lax.fori_loop(0, _NT, wbody, 0)

    tv = t_ref[...]
    c = tv[:, 1].astype(jnp.int32)
    total = jnp.float32(0.0)
    tt_row = lax.broadcasted_iota(jnp.int32, (_NT, _NT), 0)
    tt_col = lax.broadcasted_iota(jnp.int32, (_NT, _NT), 1)
    cls_iota = lax.broadcasted_iota(jnp.int32, (_NT, _NCLS), 1)
    for s in range(3):
        G = float(_NG[s])
        anch = _ANCH[s]
        rows = gath[s]
        a = a_ref[s]
        lin = lin_ref[s]
        m = (iou_ref[s] > _IOU_THRESH).astype(jnp.float32)
        mb = m > 0.5
        cnt = jnp.sum(m)
        gx = tv[:, 2] * G
        gy = tv[:, 3] * G
        gw = tv[:, 4] * G
        gh = tv[:, 5] * G
        tx = gx - jnp.floor(gx)
        ty = gy - jnp.floor(gy)
        aw = jnp.where(a == 0, anch[0][0],
                       jnp.where(a == 1, anch[1][0], anch[2][0]))
        ah = jnp.where(a == 0, anch[0][1],
                       jnp.where(a == 1, anch[1][1], anch[2][1]))
        twx = jnp.log(gw / aw)
        twy = jnp.log(gh / ah)
        sx = jax.nn.sigmoid(rows[:, 0])
        sy = jax.nn.sigmoid(rows[:, 1])
        lxy = _K * _XY * jnp.sum(
            m * ((sx - tx) ** 2 + (sy - ty) ** 2)) / (cnt * 2.0)
        lwh = _K * _WH * jnp.sum(
            m * ((rows[:, 2] - twx) ** 2 + (rows[:, 3] - twy) ** 2)) / (cnt * 2.0)
        logits = rows[:, 5:]
        mx = jnp.max(logits, axis=-1, keepdims=True)
        lse = jnp.log(jnp.sum(jnp.exp(logits - mx), axis=-1)) + mx[:, 0]
        picked = jnp.sum(jnp.where(cls_iota == c[:, None], logits, 0.0), axis=-1)
        logp = picked - lse
        lcls = _K * _CLS * (-jnp.sum(m * logp)) / cnt
        dup = jnp.any(
            (tt_col < tt_row) & mb[None, :] & (lin[None, :] == lin[:, None]),
            axis=1)
        first = mb & (~dup)
        corr = jnp.sum(jnp.where(first, rows[:, 4], 0.0))
        numel = float(_B * 3 * _NG[s] * _NG[s])
        lconf = _K * _CONF * (accs[s] - corr) / numel
        total = total + lxy + lwh + lcls + lconf
    out_ref[0, 0] = total


def kernel(p0, p1, p2, targets):
    tT = targets.T

    b, a, gj, gi, lin, iou = _sc_route(tT)

    loss = pl.pallas_call(
        _main_body,
        in_specs=[
            pl.BlockSpec(memory_space=pltpu.SMEM),
            pl.BlockSpec(memory_space=pltpu.SMEM),
            pl.BlockSpec(memory_space=pltpu.SMEM),
            pl.BlockSpec(memory_space=pltpu.SMEM),
            pl.BlockSpec(memory_space=pltpu.VMEM),
            pl.BlockSpec(memory_space=pltpu.VMEM),
            pl.BlockSpec(memory_space=pltpu.VMEM),
            pl.BlockSpec(memory_space=pltpu.VMEM),
            pl.BlockSpec(memory_space=pl.ANY),
            pl.BlockSpec(memory_space=pl.ANY),
            pl.BlockSpec(memory_space=pl.ANY),
        ],
        out_specs=pl.BlockSpec(memory_space=pltpu.SMEM),
        out_shape=jax.ShapeDtypeStruct((1, 1), jnp.float32),
        scratch_shapes=(
            [pltpu.VMEM((_B, 3, _NG[0], _NG[0], _CH), jnp.float32)]
            + [pltpu.VMEM((_P1_CB, 3, _NG[1], _NG[1], _CH), jnp.float32)
               for _ in range(4)]
            + [pltpu.VMEM((_P2_CB, 3, _NG[2], _NG[2], _CH), jnp.float32)
               for _ in range(4)]
            + [
                pltpu.VMEM((3, _NT, _CH), jnp.float32),
                pltpu.SemaphoreType.DMA,
                pltpu.SemaphoreType.DMA((4,)),
                pltpu.SemaphoreType.DMA((4,)),
                pltpu.SemaphoreType.DMA,
            ]
        ),
    )(b, a, gj, gi, targets, a, lin, iou, p0, p1, p2)

    return loss[0, 0]


# speedup vs baseline: 1.4068x; 1.4068x over previous
import functools

import jax
import jax.numpy as jnp
from jax import lax
from jax.experimental import pallas as pl
from jax.experimental.pallas import tpu as pltpu
from jax.experimental.pallas import tpu_sc as plsc

_IOU_THRESH = 0.1
_K = 10.0
_XY = 0.2
_WH = 0.1
_CLS = 0.35
_CONF = 0.35
_NG = (13, 26, 52)
_B = 16
_NCLS = 80
_ANCH = (
    ((3.625, 2.8125), (4.875, 6.1875), (11.65625, 10.1875)),
    ((1.875, 2.40625), (2.9375, 6.09375), (5.59375, 5.03125)),
    ((1.25, 1.625), (2.0, 3.75), (4.125, 2.875)),
)
_NT = 128
_CH = 85

_sc_mesh = plsc.VectorSubcoreMesh(core_axis_name="c", subcore_axis_name="s")


@functools.partial(
    pl.kernel,
    mesh=_sc_mesh,
    out_type=[
        jax.ShapeDtypeStruct((3, _NT), jnp.int32),
        jax.ShapeDtypeStruct((3, _NT), jnp.int32),
        jax.ShapeDtypeStruct((3, _NT), jnp.int32),
        jax.ShapeDtypeStruct((3, _NT), jnp.int32),
        jax.ShapeDtypeStruct((3, _NT), jnp.int32),
        jax.ShapeDtypeStruct((3, _NT), jnp.float32),
    ],
    scratch_types=[
        pltpu.VMEM((6, _NT), jnp.float32),
        pltpu.VMEM((32,), jnp.int32),
        pltpu.VMEM((32,), jnp.int32),
        pltpu.VMEM((32,), jnp.int32),
        pltpu.VMEM((32,), jnp.int32),
        pltpu.VMEM((32,), jnp.int32),
        pltpu.VMEM((32,), jnp.float32),
    ],
)
def _sc_route(tT_hbm, b_out, a_out, gj_out, gi_out, lin_out, iou_out,
              tT_v, b_v, a_v, gj_v, gi_v, lin_v, iou_v):
    wid = lax.axis_index("s") * 2 + lax.axis_index("c")
    j0 = (wid % 4) * 32
    for sc in range(3):
        @pl.when(wid // 4 == sc)
        def _(sc=sc):
            G = float(_NG[sc])
            Gi = _NG[sc]
            anch = _ANCH[sc]
            pltpu.sync_copy(tT_hbm, tT_v)
            for chunk in range(2):
                base = j0 + chunk * 16
                bf = tT_v[0, pl.ds(base, 16)]
                xf = tT_v[2, pl.ds(base, 16)]
                yf = tT_v[3, pl.ds(base, 16)]
                wf = tT_v[4, pl.ds(base, 16)]
                hf = tT_v[5, pl.ds(base, 16)]
                gw = wf * G
                gh = hf * G
                ious = []
                for k in range(3):
                    aw, ah = anch[k]
                    inter = jnp.minimum(aw, gw) * jnp.minimum(ah, gh)
                    union = aw * ah + gw * gh - inter + 1e-16
                    ious.append(inter / union)
                a16 = jnp.where(ious[1] > ious[0], 1, 0).astype(jnp.int32)
                best = jnp.maximum(ious[0], ious[1])
                a16 = jnp.where(ious[2] > best, 2, a16)
                best = jnp.maximum(best, ious[2])
                b16 = bf.astype(jnp.int32)
                gi16 = (xf * G).astype(jnp.int32)
                gj16 = (yf * G).astype(jnp.int32)
                dst = pl.ds(chunk * 16, 16)
                b_v[dst] = b16
                a_v[dst] = a16
                gj_v[dst] = gj16
                gi_v[dst] = gi16
                lin_v[dst] = (((b16 * 3 + a16) * Gi + gj16) * Gi) + gi16
                iou_v[dst] = best
            pltpu.sync_copy(b_v, b_out.at[sc, pl.ds(j0, 32)])
            pltpu.sync_copy(a_v, a_out.at[sc, pl.ds(j0, 32)])
            pltpu.sync_copy(gj_v, gj_out.at[sc, pl.ds(j0, 32)])
            pltpu.sync_copy(gi_v, gi_out.at[sc, pl.ds(j0, 32)])
            pltpu.sync_copy(lin_v, lin_out.at[sc, pl.ds(j0, 32)])
            pltpu.sync_copy(iou_v, iou_out.at[sc, pl.ds(j0, 32)])


def _softplus_sum(x):
    return jnp.sum(jnp.maximum(x, 0.0) + jnp.log1p(jnp.exp(-jnp.abs(x))))


_P1_CB = 4
_P2_CB = 1
_P2_NC = _B // _P2_CB


def _main_body(bS_ref, aS_ref, gjS_ref, giS_ref,
               t_ref, a_ref, lin_ref, iou_ref,
               p0h, p1h, p2h, out_ref,
               v0, v1a, v1b, v1c, v1d, v2a, v2b, v2c, v2d,
               x0, x1, x2, gath, sem0, sem1, sem2, semr):
    ph = (p0h, p1h, p2h)
    bufs1 = (v1a, v1b, v1c, v1d)
    bufs2 = (v2a, v2b, v2c, v2d)

    def p1_copy(k, slot):
        return pltpu.make_async_copy(
            p1h.at[pl.ds(k * _P1_CB, _P1_CB)], bufs1[slot], sem1.at[slot])

    def p2_copy(k, slot):
        return pltpu.make_async_copy(
            p2h.at[pl.ds(k * _P2_CB, _P2_CB)], bufs2[slot], sem2.at[slot])

    def row_copy(s, t):
        return pltpu.make_async_copy(
            ph[s].at[bS_ref[s, t], aS_ref[s, t], gjS_ref[s, t],
                     pl.ds(giS_ref[s, t], 1), :],
            gath.at[s, pl.ds(t, 1), :],
            semr)

    for j in range(4):
        p2_copy(j, j).start()
    for j in range(4):
        p1_copy(j, j).start()
    c0 = pltpu.make_async_copy(p0h, v0, sem0)
    c0.start()

    for s in range(3):
        def rbody(t, _, s=s):
            row_copy(s, t).start()
            return 0
        lax.fori_loop(0, _NT, rbody, 0)

    for k in range(_P2_NC):
        p2_copy(k, k % 4).wait()
        x2[pl.ds(k * _P2_CB, _P2_CB)] = bufs2[k % 4][..., 4]
        if k + 4 < _P2_NC:
            p2_copy(k + 4, k % 4).start()

    for k in range(4):
        p1_copy(k, k).wait()
        x1[pl.ds(k * _P1_CB, _P1_CB)] = bufs1[k][..., 4]

    c0.wait()
    x0[...] = v0[..., 4]

    accs = [_softplus_sum(x0[...]), _softplus_sum(x1[...]),
            _softplus_sum(x2[...])]

    for s in range(3):
        def wbody(t, _, s=s):
            row_copy(s, t).wait()
            return 0
        lax.fori_loop(0, _NT, wbody, 0)

    tv = t_ref[...]
    c = tv[:, 1].astype(jnp.int32)
    total = jnp.float32(0.0)
    tt_row = lax.broadcasted_iota(jnp.int32, (_NT, _NT), 0)
    tt_col = lax.broadcasted_iota(jnp.int32, (_NT, _NT), 1)
    cls_iota = lax.broadcasted_iota(jnp.int32, (_NT, _NCLS), 1)
    for s in range(3):
        G = float(_NG[s])
        anch = _ANCH[s]
        rows = gath[s]
        a = a_ref[s]
        lin = lin_ref[s]
        m = (iou_ref[s] > _IOU_THRESH).astype(jnp.float32)
        mb = m > 0.5
        cnt = jnp.sum(m)
        gx = tv[:, 2] * G
        gy = tv[:, 3] * G
        gw = tv[:, 4] * G
        gh = tv[:, 5] * G
        tx = gx - jnp.floor(gx)
        ty = gy - jnp.floor(gy)
        aw = jnp.where(a == 0, anch[0][0],
                       jnp.where(a == 1, anch[1][0], anch[2][0]))
        ah = jnp.where(a == 0, anch[0][1],
                       jnp.where(a == 1, anch[1][1], anch[2][1]))
        twx = jnp.log(gw / aw)
        twy = jnp.log(gh / ah)
        sx = jax.nn.sigmoid(rows[:, 0])
        sy = jax.nn.sigmoid(rows[:, 1])
        lxy = _K * _XY * jnp.sum(
            m * ((sx - tx) ** 2 + (sy - ty) ** 2)) / (cnt * 2.0)
        lwh = _K * _WH * jnp.sum(
            m * ((rows[:, 2] - twx) ** 2 + (rows[:, 3] - twy) ** 2)) / (cnt * 2.0)
        logits = rows[:, 5:]
        mx = jnp.max(logits, axis=-1, keepdims=True)
        lse = jnp.log(jnp.sum(jnp.exp(logits - mx), axis=-1)) + mx[:, 0]
        picked = jnp.sum(jnp.where(cls_iota == c[:, None], logits, 0.0), axis=-1)
        logp = picked - lse
        lcls = _K * _CLS * (-jnp.sum(m * logp)) / cnt
        dup = jnp.any(
            (tt_col < tt_row) & mb[None, :] & (lin[None, :] == lin[:, None]),
            axis=1)
        first = mb & (~dup)
        corr = jnp.sum(jnp.where(first, rows[:, 4], 0.0))
        numel = float(_B * 3 * _NG[s] * _NG[s])
        lconf = _K * _CONF * (accs[s] - corr) / numel
        total = total + lxy + lwh + lcls + lconf
    out_ref[0, 0] = total


def kernel(p0, p1, p2, targets):
    tT = targets.T

    b, a, gj, gi, lin, iou = _sc_route(tT)

    loss = pl.pallas_call(
        _main_body,
        in_specs=[
            pl.BlockSpec(memory_space=pltpu.SMEM),
            pl.BlockSpec(memory_space=pltpu.SMEM),
            pl.BlockSpec(memory_space=pltpu.SMEM),
            pl.BlockSpec(memory_space=pltpu.SMEM),
            pl.BlockSpec(memory_space=pltpu.VMEM),
            pl.BlockSpec(memory_space=pltpu.VMEM),
            pl.BlockSpec(memory_space=pltpu.VMEM),
            pl.BlockSpec(memory_space=pltpu.VMEM),
            pl.BlockSpec(memory_space=pl.ANY),
            pl.BlockSpec(memory_space=pl.ANY),
            pl.BlockSpec(memory_space=pl.ANY),
        ],
        out_specs=pl.BlockSpec(memory_space=pltpu.SMEM),
        out_shape=jax.ShapeDtypeStruct((1, 1), jnp.float32),
        scratch_shapes=(
            [pltpu.VMEM((_B, 3, _NG[0], _NG[0], _CH), jnp.float32)]
            + [pltpu.VMEM((_P1_CB, 3, _NG[1], _NG[1], _CH), jnp.float32)
               for _ in range(4)]
            + [pltpu.VMEM((_P2_CB, 3, _NG[2], _NG[2], _CH), jnp.float32)
               for _ in range(4)]
            + [
                pltpu.VMEM((_B, 3, _NG[0], _NG[0]), jnp.float32),
                pltpu.VMEM((_B, 3, _NG[1], _NG[1]), jnp.float32),
                pltpu.VMEM((_B, 3, _NG[2], _NG[2]), jnp.float32),
                pltpu.VMEM((3, _NT, _CH), jnp.float32),
                pltpu.SemaphoreType.DMA,
                pltpu.SemaphoreType.DMA((4,)),
                pltpu.SemaphoreType.DMA((4,)),
                pltpu.SemaphoreType.DMA,
            ]
        ),
    )(b, a, gj, gi, targets, a, lin, iou, p0, p1, p2)

    return loss[0, 0]
